# baseline (speedup 1.0000x reference)
import functools

import jax
import jax.numpy as jnp
from jax import lax
from jax.experimental import pallas as pl
from jax.experimental.pallas import tpu as pltpu
from jax.experimental.pallas import tpu_sc as plsc

_VOCAB = 100000
_DIM = 32
_PAIRS = _DIM // 2
_BATCH = 16384
_NUM_NEG = 20

_NC = 2
_NS = 16
_NW = _NC * _NS
_LANES = 16

_BPW = _BATCH // _NW
_CHB = 32
_NCHUNK = _BPW // _CHB
_ROWS_CH = _CHB * _NUM_NEG
_IDXW = 128
_NIDX_CH = _ROWS_CH // _IDXW
_CROWS = _BPW // _IDXW


def _logsig(x):
    a = jnp.abs(x)
    z = jnp.exp(-a)
    s = z / (2.0 + z)
    s2 = s * s
    p = 1.0 + s2 * (0.3333333432674408 + s2 * (0.2 + s2 * 0.14285714924335480))
    return jnp.minimum(x, 0.0) - 2.0 * s * p


def _sc_body(cidx_hbm, xidx_hbm, nidx_hbm, wc_hbm, wx_hbm, out_hbm,
             cidx_v, xidx_v, nidx_v, cbuf, xbuf, nbuf0, nbuf1, nbuf2, nbuf3,
             accv, sem_cx, sem_n0, sem_n1, sem_n2, sem_n3):
    wid = lax.axis_index("s") * _NC + lax.axis_index("c")
    lane = lax.iota(jnp.int32, _LANES)
    lane20 = lane * _NUM_NEG

    pltpu.sync_copy(cidx_hbm.at[pl.ds(wid * _CROWS, _CROWS)], cidx_v)
    pltpu.sync_copy(xidx_hbm.at[pl.ds(wid * _CROWS, _CROWS)], xidx_v)

    cx_ops = []
    for r in range(_CROWS):
        cx_ops.append(pltpu.async_copy(
            wc_hbm.at[cidx_v.at[r]],
            cbuf.at[pl.ds(r * _IDXW, _IDXW)], sem_cx))
        cx_ops.append(pltpu.async_copy(
            wx_hbm.at[xidx_v.at[r]],
            xbuf.at[pl.ds(r * _IDXW, _IDXW)], sem_cx))
    pltpu.sync_copy(nidx_hbm.at[pl.ds(wid * _NCHUNK * _NIDX_CH,
                                      _NCHUNK * _NIDX_CH)], nidx_v)

    def fire_chunk(c, buf, sem):
        for j in range(_NIDX_CH):
            pltpu.async_copy(wx_hbm.at[nidx_v.at[c * _NIDX_CH + j]],
                             buf.at[pl.ds(j * _IDXW, _IDXW)], sem)

    def drain_chunk(buf, sem):
        for j in range(_NIDX_CH):
            pltpu.make_async_copy(
                wx_hbm.at[nidx_v.at[j]],
                buf.at[pl.ds(j * _IDXW, _IDXW)], sem).wait()

    nbufs = (nbuf0, nbuf1, nbuf2, nbuf3)
    sems = (sem_n0, sem_n1, sem_n2, sem_n3)
    for s in range(3):
        fire_chunk(s, nbufs[s], sems[s])
    for op in cx_ops:
        op.wait()

    def as_bf(v):
        return plsc.bitcast(v, jnp.bfloat16)

    def unpack2(v):
        return plsc.unpack(plsc.bitcast(v, jnp.bfloat16),
                           format=plsc.PackFormat.INTERLEAVED,
                           preferred_element_type=jnp.float32)

    def group_partial(buf, c, g2):
        crow = lane + (c * _CHB + g2 * _LANES)
        nrows = [lane20 + (g2 * _LANES * _NUM_NEG + n)
                 for n in range(_NUM_NEG)]
        zero = jnp.zeros((2 * _LANES,), jnp.bfloat16)
        init = (zero,) * (1 + _NUM_NEG)

        def dbody(d, accs):
            dd = jnp.full((_LANES,), d, jnp.int32)
            cd = plsc.load_gather(cbuf, [crow, dd])
            cd16 = plsc.load_gather(cbuf, [crow, dd + _PAIRS])
            cp = plsc.pack(cd, cd16, format=plsc.PackFormat.INTERLEAVED)
            xp = as_bf(plsc.load_gather(xbuf, [crow, dd]))
            out = [accs[0] + cp * xp]
            for n in range(_NUM_NEG):
                rp = as_bf(plsc.load_gather(buf, [nrows[n], dd]))
                out.append(accs[n + 1] + cp * rp)
            return tuple(out)

        accs = lax.fori_loop(0, _PAIRS, dbody, init)

        def fin(a):
            ai, bi = plsc.unpack(a, format=plsc.PackFormat.INTERLEAVED,
                                 preferred_element_type=jnp.float32)
            return ai + bi

        part = _logsig(fin(accs[0]))
        for n in range(_NUM_NEG):
            part = part + _logsig(-fin(accs[n + 1]))
        return part

    def chunk_quad(t, acc):
        for s in range(4):
            c = 4 * t + s
            drain_chunk(nbufs[s], sems[s])

            @pl.when(c + 3 < _NCHUNK)
            def _():
                fire_chunk(c + 3, nbufs[(s + 3) % 4], sems[(s + 3) % 4])

            acc = acc + group_partial(nbufs[s], c, 0)
            acc = acc + group_partial(nbufs[s], c, 1)
        return acc

    acc = lax.fori_loop(0, _NCHUNK // 4, chunk_quad,
                        jnp.zeros((_LANES,), jnp.float32))
    accv[...] = acc
    pltpu.sync_copy(accv, out_hbm.at[wid])


@functools.partial(jax.jit, static_argnums=())
def _run_sc(cidx, xidx, nidx, wc, wx):
    mesh = plsc.VectorSubcoreMesh(core_axis_name="c", subcore_axis_name="s")
    f = pl.kernel(
        _sc_body,
        out_type=jax.ShapeDtypeStruct((_NW, _LANES), jnp.float32),
        mesh=mesh,
        scratch_types=[
            pltpu.VMEM((_CROWS, _IDXW), jnp.int32),
            pltpu.VMEM((_CROWS, _IDXW), jnp.int32),
            pltpu.VMEM((_NCHUNK * _NIDX_CH, _IDXW), jnp.int32),
            pltpu.VMEM((_BPW, _DIM), jnp.float32),
            pltpu.VMEM((_BPW, _PAIRS), jnp.int32),
            pltpu.VMEM((_ROWS_CH, _PAIRS), jnp.int32),
            pltpu.VMEM((_ROWS_CH, _PAIRS), jnp.int32),
            pltpu.VMEM((_ROWS_CH, _PAIRS), jnp.int32),
            pltpu.VMEM((_ROWS_CH, _PAIRS), jnp.int32),
            pltpu.VMEM((_LANES,), jnp.float32),
            pltpu.SemaphoreType.DMA,
            pltpu.SemaphoreType.DMA,
            pltpu.SemaphoreType.DMA,
            pltpu.SemaphoreType.DMA,
            pltpu.SemaphoreType.DMA,
        ],
        compiler_params=pltpu.CompilerParams(needs_layout_passes=False,
                                             use_tc_tiling_on_sc=False),
    )
    return f(cidx, xidx, nidx, wc, wx)


def _pack_bf16_pairs(w):
    u = lax.bitcast_convert_type(w.astype(jnp.bfloat16), jnp.uint16)
    lo = u[:, :_PAIRS].astype(jnp.uint32)
    hi = u[:, _PAIRS:].astype(jnp.uint32)
    return lax.bitcast_convert_type(lo | (hi << 16), jnp.int32)


def kernel(center_ids, context_ids, negative_ids, W_center, W_context):
    cidx = center_ids.astype(jnp.int32).reshape(_BATCH // _IDXW, _IDXW)
    xidx = context_ids.astype(jnp.int32).reshape(_BATCH // _IDXW, _IDXW)
    nidx = negative_ids.astype(jnp.int32).reshape(
        _BATCH * _NUM_NEG // _IDXW, _IDXW)
    wc = W_center
    wx = _pack_bf16_pairs(W_context)
    partials = _run_sc(cidx, xidx, nidx, wc, wx)
    return -(jnp.sum(partials) / _BATCH)


# speedup vs baseline: 1.1846x; 1.1846x over previous
import functools

import jax
import jax.numpy as jnp
from jax import lax
from jax.experimental import pallas as pl
from jax.experimental.pallas import tpu as pltpu
from jax.experimental.pallas import tpu_sc as plsc

_VOCAB = 100000
_DIM = 32
_PAIRS = _DIM // 2
_BATCH = 16384
_NUM_NEG = 20

_NC = 2
_NS = 16
_NW = _NC * _NS
_LANES = 16

_BPW = _BATCH // _NW
_CHB = 32
_NCHUNK = _BPW // _CHB
_ROWS_CH = _CHB * _NUM_NEG
_IDXW = 128
_NIDX_CH = _ROWS_CH // _IDXW
_CROWS = _BPW // _IDXW


def _logsig(x):
    a = jnp.abs(x)
    z = jnp.exp(-a)
    s = z / (2.0 + z)
    s2 = s * s
    p = 1.0 + s2 * (0.3333333432674408 + s2 * (0.2 + s2 * 0.14285714924335480))
    return jnp.minimum(x, 0.0) - 2.0 * s * p


def _sc_body(cidx_hbm, xidx_hbm, nidx_hbm, wc_hbm, wx_hbm, out_hbm,
             cidx_v, xidx_v, nidx_v, cbuf, xbuf, nbuf0, nbuf1, accv,
             sem_cx, sem_n0, sem_n1):
    wid = lax.axis_index("s") * _NC + lax.axis_index("c")
    lane = lax.iota(jnp.int32, _LANES)
    lane20 = lane * _NUM_NEG

    pltpu.sync_copy(cidx_hbm.at[pl.ds(wid * _CROWS, _CROWS)], cidx_v)
    pltpu.sync_copy(xidx_hbm.at[pl.ds(wid * _CROWS, _CROWS)], xidx_v)

    cx_ops = []
    for r in range(_CROWS):
        cx_ops.append(pltpu.async_copy(
            wc_hbm.at[cidx_v.at[r]],
            cbuf.at[pl.ds(r * _IDXW, _IDXW)], sem_cx))
        cx_ops.append(pltpu.async_copy(
            wx_hbm.at[xidx_v.at[r]],
            xbuf.at[pl.ds(r * _IDXW, _IDXW)], sem_cx))
    pltpu.sync_copy(nidx_hbm.at[pl.ds(wid * _NCHUNK * _NIDX_CH,
                                      _NCHUNK * _NIDX_CH)], nidx_v)

    def fire_chunk(c, buf, sem):
        for j in range(_NIDX_CH):
            pltpu.async_copy(wx_hbm.at[nidx_v.at[c * _NIDX_CH + j]],
                             buf.at[pl.ds(j * _IDXW, _IDXW)], sem)

    def drain_chunk(buf, sem):
        for j in range(_NIDX_CH):
            pltpu.make_async_copy(
                wx_hbm.at[nidx_v.at[j]],
                buf.at[pl.ds(j * _IDXW, _IDXW)], sem).wait()

    fire_chunk(0, nbuf0, sem_n0)
    for op in cx_ops:
        op.wait()

    def as_bf(v):
        return plsc.bitcast(v, jnp.bfloat16)

    def unpack2(v):
        return plsc.unpack(plsc.bitcast(v, jnp.bfloat16),
                           format=plsc.PackFormat.INTERLEAVED,
                           preferred_element_type=jnp.float32)

    def group_partial(buf, c, g2):
        crow = lane + (c * _CHB + g2 * _LANES)
        nrows = [lane20 + (g2 * _LANES * _NUM_NEG + n)
                 for n in range(_NUM_NEG)]
        zero = jnp.zeros((2 * _LANES,), jnp.bfloat16)
        init = (zero,) * (1 + _NUM_NEG)

        def dbody(d, accs):
            dd = jnp.full((_LANES,), d, jnp.int32)
            cd = plsc.load_gather(cbuf, [crow, dd])
            cd16 = plsc.load_gather(cbuf, [crow, dd + _PAIRS])
            cp = plsc.pack(cd, cd16, format=plsc.PackFormat.INTERLEAVED)
            xp = as_bf(plsc.load_gather(xbuf, [crow, dd]))
            out = [accs[0] + cp * xp]
            for n in range(_NUM_NEG):
                rp = as_bf(plsc.load_gather(buf, [nrows[n], dd]))
                out.append(accs[n + 1] + cp * rp)
            return tuple(out)

        accs = lax.fori_loop(0, _PAIRS, dbody, init)

        def fin(a):
            ai, bi = plsc.unpack(a, format=plsc.PackFormat.INTERLEAVED,
                                 preferred_element_type=jnp.float32)
            return ai + bi

        part = _logsig(fin(accs[0]))
        for n in range(_NUM_NEG):
            part = part + _logsig(-fin(accs[n + 1]))
        return part

    def chunk_pair(t, acc):
        c0 = 2 * t
        drain_chunk(nbuf0, sem_n0)
        fire_chunk(c0 + 1, nbuf1, sem_n1)
        acc = acc + group_partial(nbuf0, c0, 0)
        acc = acc + group_partial(nbuf0, c0, 1)
        drain_chunk(nbuf1, sem_n1)

        @pl.when(t < _NCHUNK // 2 - 1)
        def _():
            fire_chunk(c0 + 2, nbuf0, sem_n0)

        acc = acc + group_partial(nbuf1, c0 + 1, 0)
        acc = acc + group_partial(nbuf1, c0 + 1, 1)
        return acc

    acc = lax.fori_loop(0, _NCHUNK // 2, chunk_pair,
                        jnp.zeros((_LANES,), jnp.float32))
    accv[...] = acc
    pltpu.sync_copy(accv, out_hbm.at[wid])


@functools.partial(jax.jit, static_argnums=())
def _run_sc(cidx, xidx, nidx, wc, wx):
    mesh = plsc.VectorSubcoreMesh(core_axis_name="c", subcore_axis_name="s")
    f = pl.kernel(
        _sc_body,
        out_type=jax.ShapeDtypeStruct((_NW, _LANES), jnp.float32),
        mesh=mesh,
        scratch_types=[
            pltpu.VMEM((_CROWS, _IDXW), jnp.int32),
            pltpu.VMEM((_CROWS, _IDXW), jnp.int32),
            pltpu.VMEM((_NCHUNK * _NIDX_CH, _IDXW), jnp.int32),
            pltpu.VMEM((_BPW, _DIM), jnp.float32),
            pltpu.VMEM((_BPW, _PAIRS), jnp.int32),
            pltpu.VMEM((_ROWS_CH, _PAIRS), jnp.int32),
            pltpu.VMEM((_ROWS_CH, _PAIRS), jnp.int32),
            pltpu.VMEM((_LANES,), jnp.float32),
            pltpu.SemaphoreType.DMA,
            pltpu.SemaphoreType.DMA,
            pltpu.SemaphoreType.DMA,
        ],
        compiler_params=pltpu.CompilerParams(needs_layout_passes=False,
                                             use_tc_tiling_on_sc=False),
    )
    return f(cidx, xidx, nidx, wc, wx)


_VPW = _VOCAB // _NW
_PCH = 625
_NPCH = _VPW // _PCH


def _pack_body(w_hbm, out_hbm, bin0, bin1, bout, sem0, sem1):
    wid = lax.axis_index("s") * _NC + lax.axis_index("c")
    base = wid * _VPW
    op0 = pltpu.async_copy(w_hbm.at[pl.ds(base, _PCH)], bin0, sem0)
    bins = (bin0, bin1)
    sems = (sem0, sem1)

    for c in range(_NPCH):
        if c == 0:
            op0.wait()
        else:
            pltpu.make_async_copy(w_hbm.at[pl.ds(base, _PCH)],
                                  bins[c % 2], sems[c % 2]).wait()
        if c + 1 < _NPCH:
            pltpu.async_copy(w_hbm.at[pl.ds(base + (c + 1) * _PCH, _PCH)],
                             bins[(c + 1) % 2], sems[(c + 1) % 2])
        bin_c = bins[c % 2]

        def rbody(r, carry, bin_c=bin_c):
            a = bin_c[r, pl.ds(0, _PAIRS)]
            b = bin_c[r, pl.ds(_PAIRS, _PAIRS)]
            p = plsc.pack(a, b, format=plsc.PackFormat.INTERLEAVED)
            bout[r, ...] = plsc.bitcast(p, jnp.int32)
            return carry

        lax.fori_loop(0, _PCH, rbody, 0)
        pltpu.sync_copy(bout, out_hbm.at[pl.ds(base + c * _PCH, _PCH)])


@jax.jit
def _run_pack(w):
    mesh = plsc.VectorSubcoreMesh(core_axis_name="c", subcore_axis_name="s")
    f = pl.kernel(
        _pack_body,
        out_type=jax.ShapeDtypeStruct((_VOCAB, _PAIRS), jnp.int32),
        mesh=mesh,
        scratch_types=[
            pltpu.VMEM((_PCH, _DIM), jnp.float32),
            pltpu.VMEM((_PCH, _DIM), jnp.float32),
            pltpu.VMEM((_PCH, _PAIRS), jnp.int32),
            pltpu.SemaphoreType.DMA,
            pltpu.SemaphoreType.DMA,
        ],
        compiler_params=pltpu.CompilerParams(needs_layout_passes=False,
                                             use_tc_tiling_on_sc=False),
    )
    return f(w)


def kernel(center_ids, context_ids, negative_ids, W_center, W_context):
    cidx = center_ids.astype(jnp.int32).reshape(_BATCH // _IDXW, _IDXW)
    xidx = context_ids.astype(jnp.int32).reshape(_BATCH // _IDXW, _IDXW)
    nidx = negative_ids.astype(jnp.int32).reshape(
        _BATCH * _NUM_NEG // _IDXW, _IDXW)
    wc = W_center
    wx = _run_pack(W_context)
    partials = _run_sc(cidx, xidx, nidx, wc, wx)
    return -(jnp.sum(partials) / _BATCH)


# speedup vs baseline: 1.1849x; 1.0003x over previous
import functools

import jax
import jax.numpy as jnp
from jax import lax
from jax.experimental import pallas as pl
from jax.experimental.pallas import tpu as pltpu
from jax.experimental.pallas import tpu_sc as plsc

_VOCAB = 100000
_DIM = 32
_PAIRS = _DIM // 2
_BATCH = 16384
_NUM_NEG = 20

_NC = 2
_NS = 16
_NW = _NC * _NS
_LANES = 16

_BPW = _BATCH // _NW
_CHB = 32
_NCHUNK = _BPW // _CHB
_ROWS_CH = _CHB * _NUM_NEG
_IDXW = 128
_NIDX_CH = _ROWS_CH // _IDXW
_CROWS = _BPW // _IDXW


def _logsig(x):
    a = jnp.abs(x)
    z = jnp.exp(-a)
    s = z / (2.0 + z)
    s2 = s * s
    p = 1.0 + s2 * (0.3333333432674408 + s2 * (0.2 + s2 * 0.14285714924335480))
    return jnp.minimum(x, 0.0) - 2.0 * s * p


def _sc_body(cidx_hbm, xidx_hbm, nidx_hbm, wc_hbm, wx_hbm, out_hbm,
             cidx_v, xidx_v, nidx_v, cbuf, xbuf, nbuf0, nbuf1, accv,
             sem_cx, sem_n0, sem_n1):
    wid = lax.axis_index("s") * _NC + lax.axis_index("c")
    lane = lax.iota(jnp.int32, _LANES)
    lane20 = lane * _NUM_NEG

    pltpu.sync_copy(cidx_hbm.at[pl.ds(wid * _CROWS, _CROWS)], cidx_v)
    pltpu.sync_copy(xidx_hbm.at[pl.ds(wid * _CROWS, _CROWS)], xidx_v)

    cx_ops = []
    for r in range(_CROWS):
        cx_ops.append(pltpu.async_copy(
            wc_hbm.at[cidx_v.at[r]],
            cbuf.at[pl.ds(r * _IDXW, _IDXW)], sem_cx))
        cx_ops.append(pltpu.async_copy(
            wx_hbm.at[xidx_v.at[r]],
            xbuf.at[pl.ds(r * _IDXW, _IDXW)], sem_cx))
    pltpu.sync_copy(nidx_hbm.at[pl.ds(wid * _NCHUNK * _NIDX_CH,
                                      _NCHUNK * _NIDX_CH)], nidx_v)

    def fire_chunk(c, buf, sem):
        for j in range(_NIDX_CH):
            pltpu.async_copy(wx_hbm.at[nidx_v.at[c * _NIDX_CH + j]],
                             buf.at[pl.ds(j * _IDXW, _IDXW)], sem)

    def drain_chunk(buf, sem):
        for j in range(_NIDX_CH):
            pltpu.make_async_copy(
                wx_hbm.at[nidx_v.at[j]],
                buf.at[pl.ds(j * _IDXW, _IDXW)], sem).wait()

    fire_chunk(0, nbuf0, sem_n0)
    for op in cx_ops:
        op.wait()

    def as_bf(v):
        return plsc.bitcast(v, jnp.bfloat16)

    def unpack2(v):
        return plsc.unpack(plsc.bitcast(v, jnp.bfloat16),
                           format=plsc.PackFormat.INTERLEAVED,
                           preferred_element_type=jnp.float32)

    def group_partial(buf, c, g2):
        crow = lane + (c * _CHB + g2 * _LANES)
        nrows = [lane20 + (g2 * _LANES * _NUM_NEG + n)
                 for n in range(_NUM_NEG)]
        zero = jnp.zeros((2 * _LANES,), jnp.bfloat16)
        init = (zero,) * (1 + _NUM_NEG)

        def dbody(d, accs):
            dd = jnp.full((_LANES,), d, jnp.int32)
            cd = plsc.load_gather(cbuf, [crow, dd])
            cd16 = plsc.load_gather(cbuf, [crow, dd + _PAIRS])
            cp = plsc.pack(cd, cd16, format=plsc.PackFormat.INTERLEAVED)
            xp = as_bf(plsc.load_gather(xbuf, [crow, dd]))
            out = [accs[0] + cp * xp]
            for n in range(_NUM_NEG):
                rp = as_bf(plsc.load_gather(buf, [nrows[n], dd]))
                out.append(accs[n + 1] + cp * rp)
            return tuple(out)

        accs = lax.fori_loop(0, _PAIRS, dbody, init)

        def fin(a):
            ai, bi = plsc.unpack(a, format=plsc.PackFormat.INTERLEAVED,
                                 preferred_element_type=jnp.float32)
            return ai + bi

        part = _logsig(fin(accs[0]))
        for n in range(_NUM_NEG):
            part = part + _logsig(-fin(accs[n + 1]))
        return part

    def chunk_pair(t, acc):
        c0 = 2 * t
        drain_chunk(nbuf0, sem_n0)
        fire_chunk(c0 + 1, nbuf1, sem_n1)
        acc = acc + group_partial(nbuf0, c0, 0)
        acc = acc + group_partial(nbuf0, c0, 1)
        drain_chunk(nbuf1, sem_n1)

        @pl.when(t < _NCHUNK // 2 - 1)
        def _():
            fire_chunk(c0 + 2, nbuf0, sem_n0)

        acc = acc + group_partial(nbuf1, c0 + 1, 0)
        acc = acc + group_partial(nbuf1, c0 + 1, 1)
        return acc

    acc = lax.fori_loop(0, _NCHUNK // 2, chunk_pair,
                        jnp.zeros((_LANES,), jnp.float32))
    accv[...] = acc
    pltpu.sync_copy(accv, out_hbm.at[wid])


@functools.partial(jax.jit, static_argnums=())
def _run_sc(cidx, xidx, nidx, wc, wx):
    mesh = plsc.VectorSubcoreMesh(core_axis_name="c", subcore_axis_name="s")
    f = pl.kernel(
        _sc_body,
        out_type=jax.ShapeDtypeStruct((_NW, _LANES), jnp.float32),
        mesh=mesh,
        scratch_types=[
            pltpu.VMEM((_CROWS, _IDXW), jnp.int32),
            pltpu.VMEM((_CROWS, _IDXW), jnp.int32),
            pltpu.VMEM((_NCHUNK * _NIDX_CH, _IDXW), jnp.int32),
            pltpu.VMEM((_BPW, _DIM), jnp.float32),
            pltpu.VMEM((_BPW, _PAIRS), jnp.int32),
            pltpu.VMEM((_ROWS_CH, _PAIRS), jnp.int32),
            pltpu.VMEM((_ROWS_CH, _PAIRS), jnp.int32),
            pltpu.VMEM((_LANES,), jnp.float32),
            pltpu.SemaphoreType.DMA,
            pltpu.SemaphoreType.DMA,
            pltpu.SemaphoreType.DMA,
        ],
        compiler_params=pltpu.CompilerParams(needs_layout_passes=False,
                                             use_tc_tiling_on_sc=False),
    )
    return f(cidx, xidx, nidx, wc, wx)


_VPW = _VOCAB // _NW
_PCH = 625
_NPCH = _VPW // _PCH


def _pack_body(w_hbm, out_hbm, bin0, bin1, bout, sem0, sem1):
    wid = lax.axis_index("s") * _NC + lax.axis_index("c")
    base = wid * _VPW
    fbase = base * _DIM
    fch = _PCH * _DIM
    op0 = pltpu.async_copy(w_hbm.at[pl.ds(fbase, fch)], bin0, sem0)
    bins = (bin0, bin1)
    sems = (sem0, sem1)

    for c in range(_NPCH):
        if c == 0:
            op0.wait()
        else:
            pltpu.make_async_copy(w_hbm.at[pl.ds(fbase, fch)],
                                  bins[c % 2], sems[c % 2]).wait()
        if c + 1 < _NPCH:
            pltpu.async_copy(w_hbm.at[pl.ds(fbase + (c + 1) * fch, fch)],
                             bins[(c + 1) % 2], sems[(c + 1) % 2])
        bin_c = bins[c % 2]

        def rbody(r, carry, bin_c=bin_c):
            a = bin_c[pl.ds(r * _DIM, _PAIRS)]
            b = bin_c[pl.ds(r * _DIM + _PAIRS, _PAIRS)]
            p = plsc.pack(a, b, format=plsc.PackFormat.INTERLEAVED)
            bout[r, ...] = plsc.bitcast(p, jnp.int32)
            return carry

        lax.fori_loop(0, _PCH, rbody, 0)
        pltpu.sync_copy(bout, out_hbm.at[pl.ds(base + c * _PCH, _PCH)])


@jax.jit
def _run_pack(w):
    mesh = plsc.VectorSubcoreMesh(core_axis_name="c", subcore_axis_name="s")
    f = pl.kernel(
        _pack_body,
        out_type=jax.ShapeDtypeStruct((_VOCAB, _PAIRS), jnp.int32),
        mesh=mesh,
        scratch_types=[
            pltpu.VMEM((_PCH * _DIM,), jnp.float32),
            pltpu.VMEM((_PCH * _DIM,), jnp.float32),
            pltpu.VMEM((_PCH, _PAIRS), jnp.int32),
            pltpu.SemaphoreType.DMA,
            pltpu.SemaphoreType.DMA,
        ],
        compiler_params=pltpu.CompilerParams(needs_layout_passes=False,
                                             use_tc_tiling_on_sc=False),
    )
    return f(w)


def kernel(center_ids, context_ids, negative_ids, W_center, W_context):
    cidx = center_ids.astype(jnp.int32).reshape(_BATCH // _IDXW, _IDXW)
    xidx = context_ids.astype(jnp.int32).reshape(_BATCH // _IDXW, _IDXW)
    nidx = negative_ids.astype(jnp.int32).reshape(
        _BATCH * _NUM_NEG // _IDXW, _IDXW)
    wc = W_center
    wx = _run_pack(W_context.reshape(-1))
    partials = _run_sc(cidx, xidx, nidx, wc, wx)
    return -(jnp.sum(partials) / _BATCH)


# speedup vs baseline: 1.1885x; 1.0030x over previous
import functools

import jax
import jax.numpy as jnp
from jax import lax
from jax.experimental import pallas as pl
from jax.experimental.pallas import tpu as pltpu
from jax.experimental.pallas import tpu_sc as plsc

_VOCAB = 100000
_DIM = 32
_PAIRS = _DIM // 2
_BATCH = 16384
_NUM_NEG = 20

_NC = 2
_NS = 16
_NW = _NC * _NS
_LANES = 16

_BPW = _BATCH // _NW
_CHB = 32
_NCHUNK = _BPW // _CHB
_ROWS_CH = _CHB * _NUM_NEG
_IDXW = 128
_NIDX_CH = _ROWS_CH // _IDXW
_CROWS = _BPW // _IDXW


def _logsig(x):
    a = jnp.abs(x)
    z = jnp.exp(-a)
    s = z / (2.0 + z)
    s2 = s * s
    p = 1.0 + s2 * (0.3333333432674408 + s2 * (0.2 + s2 * 0.14285714924335480))
    return jnp.minimum(x, 0.0) - 2.0 * s * p


def _sc_body(cidx_hbm, xidx_hbm, nidx_hbm, wc_hbm, wx_hbm, out_hbm,
             cidx_v, xidx_v, nidx_v, cbuf, xbuf, nbuf0, nbuf1, accv,
             sem_cx, sem_n0, sem_n1):
    wid = lax.axis_index("s") * _NC + lax.axis_index("c")
    lane = lax.iota(jnp.int32, _LANES)
    lane20 = lane * _NUM_NEG

    pltpu.sync_copy(cidx_hbm.at[pl.ds(wid * _CROWS, _CROWS)], cidx_v)
    pltpu.sync_copy(xidx_hbm.at[pl.ds(wid * _CROWS, _CROWS)], xidx_v)

    cx_ops = []
    for r in range(_CROWS):
        cx_ops.append(pltpu.async_copy(
            wc_hbm.at[cidx_v.at[r]],
            cbuf.at[pl.ds(r * _IDXW, _IDXW)], sem_cx))
        cx_ops.append(pltpu.async_copy(
            wx_hbm.at[xidx_v.at[r]],
            xbuf.at[pl.ds(r * _IDXW, _IDXW)], sem_cx))
    pltpu.sync_copy(nidx_hbm.at[pl.ds(wid * _NCHUNK * _NIDX_CH,
                                      _NCHUNK * _NIDX_CH)], nidx_v)

    def fire_chunk(c, buf, sem):
        for j in range(_NIDX_CH):
            pltpu.async_copy(wx_hbm.at[nidx_v.at[c * _NIDX_CH + j]],
                             buf.at[pl.ds(j * _IDXW, _IDXW)], sem)

    def drain_chunk(buf, sem):
        for j in range(_NIDX_CH):
            pltpu.make_async_copy(
                wx_hbm.at[nidx_v.at[j]],
                buf.at[pl.ds(j * _IDXW, _IDXW)], sem).wait()

    fire_chunk(0, nbuf0, sem_n0)
    for op in cx_ops:
        op.wait()

    def as_bf(v):
        return plsc.bitcast(v, jnp.bfloat16)

    def group_partial(buf, c, g2):
        crow = lane + (c * _CHB + g2 * _LANES)
        nrows = [lane20 + (g2 * _LANES * _NUM_NEG + n)
                 for n in range(_NUM_NEG)]
        zero = jnp.zeros((2 * _LANES,), jnp.bfloat16)
        init = (zero,) * (1 + _NUM_NEG)

        def dbody(d, accs):
            dd = jnp.full((_LANES,), d, jnp.int32)
            cd = plsc.load_gather(cbuf, [crow, dd])
            cd16 = plsc.load_gather(cbuf, [crow, dd + _PAIRS])
            cp = plsc.pack(cd, cd16, format=plsc.PackFormat.INTERLEAVED)
            xp = as_bf(plsc.load_gather(xbuf, [crow, dd]))
            out = [accs[0] + cp * xp]
            for n in range(_NUM_NEG):
                rp = as_bf(plsc.load_gather(buf, [nrows[n], dd]))
                out.append(accs[n + 1] + cp * rp)
            return tuple(out)

        accs = lax.fori_loop(0, _PAIRS, dbody, init)

        def fin(a):
            ai, bi = plsc.unpack(a, format=plsc.PackFormat.INTERLEAVED,
                                 preferred_element_type=jnp.float32)
            return ai + bi

        part = _logsig(fin(accs[0]))
        for n in range(_NUM_NEG):
            part = part + _logsig(-fin(accs[n + 1]))
        return part

    def chunk_pair(t, acc):
        c0 = 2 * t
        drain_chunk(nbuf0, sem_n0)
        fire_chunk(c0 + 1, nbuf1, sem_n1)
        acc = acc + group_partial(nbuf0, c0, 0)
        acc = acc + group_partial(nbuf0, c0, 1)
        drain_chunk(nbuf1, sem_n1)

        @pl.when(t < _NCHUNK // 2 - 1)
        def _():
            fire_chunk(c0 + 2, nbuf0, sem_n0)

        acc = acc + group_partial(nbuf1, c0 + 1, 0)
        acc = acc + group_partial(nbuf1, c0 + 1, 1)
        return acc

    acc = lax.fori_loop(0, _NCHUNK // 2, chunk_pair,
                        jnp.zeros((_LANES,), jnp.float32))
    accv[...] = acc
    pltpu.sync_copy(accv, out_hbm.at[wid])


@functools.partial(jax.jit, static_argnums=())
def _run_sc(cidx, xidx, nidx, wc, wx):
    mesh = plsc.VectorSubcoreMesh(core_axis_name="c", subcore_axis_name="s")
    f = pl.kernel(
        _sc_body,
        out_type=jax.ShapeDtypeStruct((_NW, _LANES), jnp.float32),
        mesh=mesh,
        scratch_types=[
            pltpu.VMEM((_CROWS, _IDXW), jnp.int32),
            pltpu.VMEM((_CROWS, _IDXW), jnp.int32),
            pltpu.VMEM((_NCHUNK * _NIDX_CH, _IDXW), jnp.int32),
            pltpu.VMEM((_BPW, _DIM), jnp.float32),
            pltpu.VMEM((_BPW, _PAIRS), jnp.int32),
            pltpu.VMEM((_ROWS_CH, _PAIRS), jnp.int32),
            pltpu.VMEM((_ROWS_CH, _PAIRS), jnp.int32),
            pltpu.VMEM((_LANES,), jnp.float32),
            pltpu.SemaphoreType.DMA,
            pltpu.SemaphoreType.DMA,
            pltpu.SemaphoreType.DMA,
        ],
        compiler_params=pltpu.CompilerParams(needs_layout_passes=False,
                                             use_tc_tiling_on_sc=False),
    )
    return f(cidx, xidx, nidx, wc, wx)


_VPW = _VOCAB // _NW
_PCH = 625
_NPCH = _VPW // _PCH


def _pack_body(w_hbm, out_hbm, bin0, bin1, bout, sem0, sem1):
    wid = lax.axis_index("s") * _NC + lax.axis_index("c")
    base = wid * _VPW
    fbase = base * _DIM
    fch = _PCH * _DIM
    op0 = pltpu.async_copy(w_hbm.at[pl.ds(fbase, fch)], bin0, sem0)
    bins = (bin0, bin1)
    sems = (sem0, sem1)

    for c in range(_NPCH):
        if c == 0:
            op0.wait()
        else:
            pltpu.make_async_copy(w_hbm.at[pl.ds(fbase, fch)],
                                  bins[c % 2], sems[c % 2]).wait()
        if c + 1 < _NPCH:
            pltpu.async_copy(w_hbm.at[pl.ds(fbase + (c + 1) * fch, fch)],
                             bins[(c + 1) % 2], sems[(c + 1) % 2])
        bin_c = bins[c % 2]

        def rbody(r, carry, bin_c=bin_c):
            a = bin_c[pl.ds(r * _DIM, _PAIRS)]
            b = bin_c[pl.ds(r * _DIM + _PAIRS, _PAIRS)]
            p = plsc.pack(a, b, format=plsc.PackFormat.INTERLEAVED)
            bout[r, ...] = plsc.bitcast(p, jnp.int32)
            return carry

        lax.fori_loop(0, _PCH, rbody, 0)
        pltpu.sync_copy(bout, out_hbm.at[pl.ds(base + c * _PCH, _PCH)])


@jax.jit
def _run_pack(w):
    mesh = plsc.VectorSubcoreMesh(core_axis_name="c", subcore_axis_name="s")
    f = pl.kernel(
        _pack_body,
        out_type=jax.ShapeDtypeStruct((_VOCAB, _PAIRS), jnp.int32),
        mesh=mesh,
        scratch_types=[
            pltpu.VMEM((_PCH * _DIM,), jnp.float32),
            pltpu.VMEM((_PCH * _DIM,), jnp.float32),
            pltpu.VMEM((_PCH, _PAIRS), jnp.int32),
            pltpu.SemaphoreType.DMA,
            pltpu.SemaphoreType.DMA,
        ],
        compiler_params=pltpu.CompilerParams(needs_layout_passes=False,
                                             use_tc_tiling_on_sc=False),
    )
    return f(w)


def kernel(center_ids, context_ids, negative_ids, W_center, W_context):
    cidx = center_ids.astype(jnp.int32).reshape(_BATCH // _IDXW, _IDXW)
    xidx = context_ids.astype(jnp.int32).reshape(_BATCH // _IDXW, _IDXW)
    nidx = negative_ids.astype(jnp.int32).reshape(
        _BATCH * _NUM_NEG // _IDXW, _IDXW)
    wc = W_center
    wx = _run_pack(W_context.reshape(-1))
    partials = _run_sc(cidx, xidx, nidx, wc, wx)
    return -(jnp.sum(partials) / _BATCH)
